# baseline (device time: 14631 ns/iter reference)
import jax
import jax.numpy as jnp
from jax import lax
from jax.experimental import pallas as pl
from jax.experimental.pallas import tpu as pltpu

PH = 13
ROWS = 16
H = PH * ROWS


def kernel(x):
    m, n = x.shape
    d_rows = m - 2 * H

    def body(x_ref, out_ref, x_send_sems, x_recv_sems, y_send_sems,
             y_recv_sems, loc_sem):
        my_x = lax.axis_index("x")
        my_y = lax.axis_index("y")
        other_x = 1 - my_x
        other_y = 1 - my_y

        barrier_sem = pltpu.get_barrier_semaphore()
        for dev in [(other_x, my_y), (my_x, other_y)]:
            pl.semaphore_signal(
                barrier_sem, inc=1,
                device_id=dev, device_id_type=pl.DeviceIdType.MESH,
            )
        pl.semaphore_wait(barrier_sem, 2)

        loc = pltpu.make_async_copy(
            x_ref, out_ref.at[pl.ds(my_x * m, m), :], loc_sem
        )
        loc.start()

        x_sends = []
        for k in range(PH):
            row = my_y * H + k * ROWS
            rdma = pltpu.make_async_remote_copy(
                src_ref=x_ref.at[pl.ds(row, ROWS), :],
                dst_ref=out_ref.at[pl.ds(my_x * m + row, ROWS), :],
                send_sem=x_send_sems.at[k],
                recv_sem=x_recv_sems.at[k],
                device_id=(other_x, my_y),
                device_id_type=pl.DeviceIdType.MESH,
            )
            rdma.start()
            x_sends.append(rdma)
        d_send = pltpu.make_async_remote_copy(
            src_ref=x_ref.at[pl.ds(2 * H, d_rows), :],
            dst_ref=out_ref.at[pl.ds(my_x * m + 2 * H, d_rows), :],
            send_sem=x_send_sems.at[PH],
            recv_sem=x_recv_sems.at[PH],
            device_id=(other_x, my_y),
            device_id_type=pl.DeviceIdType.MESH,
        )
        d_send.start()

        y_sends = []
        for k in range(PH):
            x_sends[k].wait_recv()
            row = other_x * m + my_y * H + k * ROWS
            rdma = pltpu.make_async_remote_copy(
                src_ref=out_ref.at[pl.ds(row, ROWS), :],
                dst_ref=out_ref.at[pl.ds(row, ROWS), :],
                send_sem=y_send_sems.at[k],
                recv_sem=y_recv_sems.at[k],
                device_id=(my_x, other_y),
                device_id_type=pl.DeviceIdType.MESH,
            )
            rdma.start()
            y_sends.append(rdma)

        d_send.wait_recv()

        for k in range(PH):
            row = other_x * m + other_y * H + k * ROWS
            recv = pltpu.make_async_remote_copy(
                src_ref=out_ref.at[pl.ds(row, ROWS), :],
                dst_ref=out_ref.at[pl.ds(row, ROWS), :],
                send_sem=y_send_sems.at[k],
                recv_sem=y_recv_sems.at[k],
                device_id=(my_x, other_y),
                device_id_type=pl.DeviceIdType.MESH,
            )
            recv.wait_recv()

        loc.wait()
        d_send.wait_send()
        for k in range(PH):
            x_sends[k].wait_send()
            y_sends[k].wait_send()

    return pl.pallas_call(
        body,
        out_shape=jax.ShapeDtypeStruct((2 * m, n), x.dtype),
        in_specs=[pl.BlockSpec(memory_space=pltpu.VMEM)],
        out_specs=pl.BlockSpec(memory_space=pltpu.VMEM),
        scratch_shapes=[
            pltpu.SemaphoreType.DMA((PH + 1,)),
            pltpu.SemaphoreType.DMA((PH + 1,)),
            pltpu.SemaphoreType.DMA((PH,)),
            pltpu.SemaphoreType.DMA((PH,)),
            pltpu.SemaphoreType.DMA,
        ],
        compiler_params=pltpu.CompilerParams(collective_id=0),
    )(x)
